# baseline (speedup 1.0000x reference)
import jax
import jax.numpy as jnp
from jax import lax
from jax.experimental import pallas as pl
from jax.experimental.pallas import tpu as pltpu
from jax.experimental.pallas import tpu_sc as plsc

N = 100000
IN_DIM = 16
F_DIM = 200
NZ = 128
NC = 2
NS = 16
NW = NC * NS
ROWS_W = N // NW
CHUNK = 125
NCHUNK = ROWS_W // CHUNK
IN_CH = CHUNK * IN_DIM
OUT_CH = CHUNK * F_DIM
OUT_PAD = OUT_CH + 8
NGRP = NZ // 16


def _body(x_hbm, sel_hbm, ov_hbm, out_hbm,
          in0, in1, out0, out1, selv, ovv,
          sin0, sin1, sout0, sout1):
    wid = lax.axis_index("s") * NC + lax.axis_index("c")
    base_x = wid * (ROWS_W * IN_DIM)
    base_o = wid * (ROWS_W * F_DIM)

    pltpu.sync_copy(sel_hbm, selv)
    pltpu.sync_copy(ov_hbm, ovv)
    sel_vecs = [selv[pl.ds(16 * g, 16)] for g in range(NGRP)]
    ov_vecs = [ovv[pl.ds(16 * g, 16)] for g in range(NGRP)]

    zeros = jnp.zeros((16,), jnp.float32)

    def zbody(i, _):
        out0[pl.ds(i * 16, 16)] = zeros
        out1[pl.ds(i * 16, 16)] = zeros
        return ()

    lax.fori_loop(0, OUT_PAD // 16, zbody, ())

    in_bufs = [in0, in1]
    out_bufs = [out0, out1]
    sins = [sin0, sin1]
    souts = [sout0, sout1]

    def in_copy(c, b):
        return pltpu.make_async_copy(
            x_hbm.at[pl.ds(base_x + c * IN_CH, IN_CH)], in_bufs[b], sins[b])

    def out_copy(c, b):
        return pltpu.make_async_copy(
            out_bufs[b].at[pl.ds(0, OUT_CH)],
            out_hbm.at[pl.ds(base_o + c * OUT_CH, OUT_CH)], souts[b])

    in_copy(0, 0).start()
    in_copy(1, 1).start()

    for c in range(NCHUNK):
        b = c & 1
        in_copy(c, b).wait()
        if c >= 2:
            out_copy(c - 2, b).wait()
        ibuf = in_bufs[b]
        obuf = out_bufs[b]

        def row_body(r, _, ibuf=ibuf, obuf=obuf):
            rb = r * IN_DIM
            ob = r * F_DIM
            for g in range(NGRP):
                vals = plsc.load_gather(ibuf, [sel_vecs[g] + rb])
                obuf[pl.ds(ob + 16 * g, 16)] = vals * ov_vecs[g]
            return ()

        lax.fori_loop(0, CHUNK, row_body, ())
        out_copy(c, b).start()
        if c + 2 < NCHUNK:
            in_copy(c + 2, b).start()

    out_copy(NCHUNK - 2, (NCHUNK - 2) & 1).wait()
    out_copy(NCHUNK - 1, (NCHUNK - 1) & 1).wait()


_sc_call = pl.kernel(
    _body,
    out_type=jax.ShapeDtypeStruct((N * F_DIM,), jnp.float32),
    mesh=plsc.VectorSubcoreMesh(
        core_axis_name="c", subcore_axis_name="s",
        num_cores=NC, num_subcores=NS),
    scratch_types=[
        pltpu.VMEM((IN_CH,), jnp.float32),
        pltpu.VMEM((IN_CH,), jnp.float32),
        pltpu.VMEM((OUT_PAD,), jnp.float32),
        pltpu.VMEM((OUT_PAD,), jnp.float32),
        pltpu.VMEM((NZ,), jnp.int32),
        pltpu.VMEM((NZ,), jnp.float32),
        pltpu.SemaphoreType.DMA,
        pltpu.SemaphoreType.DMA,
        pltpu.SemaphoreType.DMA,
        pltpu.SemaphoreType.DMA,
    ],
    compiler_params=pltpu.CompilerParams(needs_layout_passes=False),
)


@jax.jit
def kernel(charge_density, overlap_constants, select_indices):
    x = charge_density.reshape(-1)
    sel = select_indices.astype(jnp.int32)
    ov = overlap_constants.astype(jnp.float32)
    out = _sc_call(x, sel, ov)
    return out.reshape(charge_density.shape[0], F_DIM)


# speedup vs baseline: 1.1597x; 1.1597x over previous
import jax
import jax.numpy as jnp
from jax import lax
from jax.experimental import pallas as pl
from jax.experimental.pallas import tpu as pltpu
from jax.experimental.pallas import tpu_sc as plsc

N = 100000
IN_DIM = 16
F_DIM = 200
NZ = 128
NC = 2
NS = 16
NW = NC * NS
ROWS_W = N // NW
CHUNK = 125
NCHUNK = ROWS_W // CHUNK
IN_CH = CHUNK * IN_DIM
OUT_CH = CHUNK * F_DIM
OUT_PAD = OUT_CH + 8
NGRP = NZ // 16


def _body(x_hbm, sel_hbm, ov_hbm, out_hbm,
          in0, in1, out0, out1, selv, ovv,
          sin0, sin1, sout0, sout1):
    wid = lax.axis_index("s") * NC + lax.axis_index("c")
    base_x = wid * (ROWS_W * IN_DIM)
    base_o = wid * (ROWS_W * F_DIM)

    pltpu.sync_copy(sel_hbm, selv)
    pltpu.sync_copy(ov_hbm, ovv)
    sel_vecs = [selv[pl.ds(16 * g, 16)] for g in range(NGRP)]
    ov_vecs = [ovv[pl.ds(16 * g, 16)] for g in range(NGRP)]

    zeros = jnp.zeros((16,), jnp.float32)

    @plsc.parallel_loop(0, OUT_PAD // 16, unroll=4)
    def _zero(i):
        out0[pl.ds(i * 16, 16)] = zeros
        out1[pl.ds(i * 16, 16)] = zeros

    in_bufs = [in0, in1]
    out_bufs = [out0, out1]
    sins = [sin0, sin1]
    souts = [sout0, sout1]

    def in_copy(c, b):
        return pltpu.make_async_copy(
            x_hbm.at[pl.ds(base_x + c * IN_CH, IN_CH)], in_bufs[b], sins[b])

    def out_copy(c, b):
        return pltpu.make_async_copy(
            out_bufs[b].at[pl.ds(0, OUT_CH)],
            out_hbm.at[pl.ds(base_o + c * OUT_CH, OUT_CH)], souts[b])

    in_copy(0, 0).start()
    in_copy(1, 1).start()

    for c in range(NCHUNK):
        b = c & 1
        in_copy(c, b).wait()
        if c >= 2:
            out_copy(c - 2, b).wait()
        ibuf = in_bufs[b]
        obuf = out_bufs[b]

        @plsc.parallel_loop(0, CHUNK, unroll=8)
        def _rows(r, ibuf=ibuf, obuf=obuf):
            row = ibuf.at[pl.ds(r * IN_DIM, IN_DIM)]
            ob = r * F_DIM
            for g in range(NGRP):
                vals = plsc.load_gather(row, [sel_vecs[g]])
                obuf[pl.ds(ob + 16 * g, 16)] = vals * ov_vecs[g]
        out_copy(c, b).start()
        if c + 2 < NCHUNK:
            in_copy(c + 2, b).start()

    out_copy(NCHUNK - 2, (NCHUNK - 2) & 1).wait()
    out_copy(NCHUNK - 1, (NCHUNK - 1) & 1).wait()


_sc_call = pl.kernel(
    _body,
    out_type=jax.ShapeDtypeStruct((N * F_DIM,), jnp.float32),
    mesh=plsc.VectorSubcoreMesh(
        core_axis_name="c", subcore_axis_name="s",
        num_cores=NC, num_subcores=NS),
    scratch_types=[
        pltpu.VMEM((IN_CH,), jnp.float32),
        pltpu.VMEM((IN_CH,), jnp.float32),
        pltpu.VMEM((OUT_PAD,), jnp.float32),
        pltpu.VMEM((OUT_PAD,), jnp.float32),
        pltpu.VMEM((NZ,), jnp.int32),
        pltpu.VMEM((NZ,), jnp.float32),
        pltpu.SemaphoreType.DMA,
        pltpu.SemaphoreType.DMA,
        pltpu.SemaphoreType.DMA,
        pltpu.SemaphoreType.DMA,
    ],
    compiler_params=pltpu.CompilerParams(needs_layout_passes=False),
)


@jax.jit
def kernel(charge_density, overlap_constants, select_indices):
    x = charge_density.reshape(-1)
    sel = select_indices.astype(jnp.int32)
    ov = overlap_constants.astype(jnp.float32)
    out = _sc_call(x, sel, ov)
    return out.reshape(charge_density.shape[0], F_DIM)


# speedup vs baseline: 3.9052x; 3.3673x over previous
import jax
import jax.numpy as jnp
from jax import lax
from jax.experimental import pallas as pl
from jax.experimental.pallas import tpu as pltpu
from jax.experimental.pallas import tpu_sc as plsc

N = 100000
IN_DIM = 16
F_DIM = 200
NZ = 128
NC = 2
NS = 16
NW = NC * NS
CHUNK = 160
NCHUNK = N // CHUNK
KMAX = -(-NCHUNK // NW)
NGRP = NZ // 16


def _body(x_hbm, sel_hbm, ov_hbm, out_hbm,
          in0, in1, out0, out1, selv, ovv,
          sin0, sin1, sout0, sout1):
    wid = lax.axis_index("s") * NC + lax.axis_index("c")

    pltpu.sync_copy(sel_hbm, selv)
    pltpu.sync_copy(ov_hbm, ovv)
    sel_vecs = [selv[pl.ds(16 * g, 16)] for g in range(NGRP)]
    ov_vecs = [ovv[pl.ds(16 * g, 16)] for g in range(NGRP)]

    zeros = jnp.zeros((16,), jnp.float32)
    lane = lax.iota(jnp.int32, 16)
    col_vecs = [lane + (16 * g) for g in range(NGRP)]
    zcol_vecs = [lane + (120 + 16 * t) for t in range(5)]

    @plsc.parallel_loop(0, CHUNK, unroll=4)
    def _zero(r):
        rvec = jnp.full((16,), r, jnp.int32)
        for t in range(5):
            plsc.store_scatter(out0, [rvec, zcol_vecs[t]], zeros)
            plsc.store_scatter(out1, [rvec, zcol_vecs[t]], zeros)

    in_bufs = [in0, in1]
    out_bufs = [out0, out1]
    sins = [sin0, sin1]
    souts = [sout0, sout1]

    def cid(k):
        return wid + NW * k

    def in_copy(k, b):
        return pltpu.make_async_copy(
            x_hbm.at[pl.ds(cid(k) * CHUNK, CHUNK)], in_bufs[b], sins[b])

    def out_copy(k, b):
        return pltpu.make_async_copy(
            out_bufs[b],
            out_hbm.at[pl.ds(cid(k) * CHUNK, CHUNK)], souts[b])

    def when_valid(k, fn):
        if (k + 1) * NW <= NCHUNK:
            fn()
        else:
            pl.when(cid(k) < NCHUNK)(fn)

    when_valid(0, lambda: in_copy(0, 0).start())
    when_valid(1, lambda: in_copy(1, 1).start())

    for k in range(KMAX):
        b = k & 1

        def step(k=k, b=b):
            in_copy(k, b).wait()
            if k >= 2:
                out_copy(k - 2, b).wait()
            ibuf = in_bufs[b]
            obuf = out_bufs[b]

            @plsc.parallel_loop(0, CHUNK, unroll=8)
            def _rows(r):
                rvec = jnp.full((16,), r, jnp.int32)
                for g in range(NGRP):
                    vals = plsc.load_gather(ibuf, [rvec, sel_vecs[g]])
                    plsc.store_scatter(obuf, [rvec, col_vecs[g]],
                                       vals * ov_vecs[g])

            out_copy(k, b).start()
            if k + 2 < KMAX:
                when_valid(k + 2, lambda: in_copy(k + 2, b).start())

        when_valid(k, step)

    when_valid(KMAX - 2, lambda: out_copy(KMAX - 2, (KMAX - 2) & 1).wait())
    when_valid(KMAX - 1, lambda: out_copy(KMAX - 1, (KMAX - 1) & 1).wait())


_sc_call = pl.kernel(
    _body,
    out_type=jax.ShapeDtypeStruct((N, F_DIM), jnp.float32),
    mesh=plsc.VectorSubcoreMesh(
        core_axis_name="c", subcore_axis_name="s",
        num_cores=NC, num_subcores=NS),
    scratch_types=[
        pltpu.VMEM((CHUNK, IN_DIM), jnp.float32),
        pltpu.VMEM((CHUNK, IN_DIM), jnp.float32),
        pltpu.VMEM((CHUNK, F_DIM), jnp.float32),
        pltpu.VMEM((CHUNK, F_DIM), jnp.float32),
        pltpu.VMEM((NZ,), jnp.int32),
        pltpu.VMEM((NZ,), jnp.float32),
        pltpu.SemaphoreType.DMA,
        pltpu.SemaphoreType.DMA,
        pltpu.SemaphoreType.DMA,
        pltpu.SemaphoreType.DMA,
    ],
    compiler_params=pltpu.CompilerParams(needs_layout_passes=False),
)


@jax.jit
def kernel(charge_density, overlap_constants, select_indices):
    sel = select_indices.astype(jnp.int32)
    ov = overlap_constants.astype(jnp.float32)
    return _sc_call(charge_density, sel, ov)


# speedup vs baseline: 5.0605x; 1.2958x over previous
import jax
import jax.numpy as jnp
from jax import lax
from jax.experimental import pallas as pl
from jax.experimental.pallas import tpu as pltpu
from jax.experimental.pallas import tpu_sc as plsc

N = 100000
IN_DIM = 16
F_DIM = 200
NZ = 128
NC = 2
NS = 16
NW = NC * NS
CHUNK = 160
NCHUNK = N // CHUNK
KMAX = -(-NCHUNK // NW)
NGRP = NZ // 16


def _body(x_hbm, sel_hbm, ov_hbm, out_hbm,
          in0, in1, out0, out1, selv, ovv,
          sin0, sin1, sout0, sout1):
    wid = lax.axis_index("s") * NC + lax.axis_index("c")

    pltpu.sync_copy(sel_hbm, selv)
    pltpu.sync_copy(ov_hbm, ovv)
    sel_vecs = [selv[pl.ds(16 * g, 16)] for g in range(NGRP)]
    ov_vecs = [ovv[pl.ds(16 * g, 16)] for g in range(NGRP)]

    lane = lax.iota(jnp.int32, 16)
    col_vecs = [lane + (16 * g) for g in range(NGRP)]

    in_bufs = [in0, in1]
    out_bufs = [out0, out1]
    sins = [sin0, sin1]
    souts = [sout0, sout1]

    def cid(k):
        return wid + NW * k

    def in_copy(k, b):
        return pltpu.make_async_copy(
            x_hbm.at[pl.ds(cid(k) * CHUNK, CHUNK)], in_bufs[b], sins[b])

    def out_copy(k, b):
        return pltpu.make_async_copy(
            out_bufs[b],
            out_hbm.at[pl.ds(cid(k) * CHUNK, CHUNK)], souts[b])

    def when_valid(k, fn):
        if (k + 1) * NW <= NCHUNK:
            fn()
        else:
            pl.when(cid(k) < NCHUNK)(fn)

    when_valid(0, lambda: in_copy(0, 0).start())
    when_valid(1, lambda: in_copy(1, 1).start())

    for k in range(KMAX):
        b = k & 1

        def step(k=k, b=b):
            in_copy(k, b).wait()
            if k >= 2:
                out_copy(k - 2, b).wait()
            ibuf = in_bufs[b]
            obuf = out_bufs[b]

            @plsc.parallel_loop(0, CHUNK, unroll=4)
            def _rows(r):
                rvec = jnp.full((16,), r, jnp.int32)
                for g in range(NGRP):
                    vals = plsc.load_gather(ibuf, [rvec, sel_vecs[g]])
                    plsc.store_scatter(obuf, [rvec, col_vecs[g]],
                                       vals * ov_vecs[g])

            out_copy(k, b).start()
            if k + 2 < KMAX:
                when_valid(k + 2, lambda: in_copy(k + 2, b).start())

        when_valid(k, step)

    when_valid(KMAX - 2, lambda: out_copy(KMAX - 2, (KMAX - 2) & 1).wait())
    when_valid(KMAX - 1, lambda: out_copy(KMAX - 1, (KMAX - 1) & 1).wait())


_sc_call = pl.kernel(
    _body,
    out_type=jax.ShapeDtypeStruct((N, NZ), jnp.float32),
    mesh=plsc.VectorSubcoreMesh(
        core_axis_name="c", subcore_axis_name="s",
        num_cores=NC, num_subcores=NS),
    scratch_types=[
        pltpu.VMEM((CHUNK, IN_DIM), jnp.float32),
        pltpu.VMEM((CHUNK, IN_DIM), jnp.float32),
        pltpu.VMEM((CHUNK, NZ), jnp.float32),
        pltpu.VMEM((CHUNK, NZ), jnp.float32),
        pltpu.VMEM((NZ,), jnp.int32),
        pltpu.VMEM((NZ,), jnp.float32),
        pltpu.SemaphoreType.DMA,
        pltpu.SemaphoreType.DMA,
        pltpu.SemaphoreType.DMA,
        pltpu.SemaphoreType.DMA,
    ],
    compiler_params=pltpu.CompilerParams(needs_layout_passes=False),
)


NBT = 2048


def _fin_body(sc_ref, o_ref):
    o_ref[0:NZ, :] = sc_ref[...].T
    o_ref[NZ:F_DIM, :] = jnp.zeros((F_DIM - NZ, NBT), jnp.float32)


_fin = pl.pallas_call(
    _fin_body,
    grid=(-(-N // NBT),),
    in_specs=[pl.BlockSpec((NBT, NZ), lambda i: (i, 0))],
    out_specs=pl.BlockSpec((F_DIM, NBT), lambda i: (0, i)),
    out_shape=jax.ShapeDtypeStruct((F_DIM, N), jnp.float32),
)


@jax.jit
def kernel(charge_density, overlap_constants, select_indices):
    sel = select_indices.astype(jnp.int32)
    ov = overlap_constants.astype(jnp.float32)
    scaled = _sc_call(charge_density, sel, ov)
    return _fin(scaled).T


# speedup vs baseline: 7.3821x; 1.4588x over previous
import jax
import jax.numpy as jnp
from jax import lax
from jax.experimental import pallas as pl
from jax.experimental.pallas import tpu as pltpu
from jax.experimental.pallas import tpu_sc as plsc

N = 100000
IN_DIM = 16
F_DIM = 200
NZ = 128
NC = 2
NS = 16
NW = NC * NS

NBT = 1280
NBLK_A = 53
NA_ROWS = NBLK_A * NBT
NB_ROWS = N - NA_ROWS
NBLK_B = -(-NB_ROWS // NBT)

CHUNK = 160
NCHUNK = NB_ROWS // CHUNK
KMAX = -(-NCHUNK // NW)
NGRP = NZ // 16


def _sc_body(x_hbm, sel_hbm, ov_hbm, out_hbm,
             in0, in1, out0, out1, selv, ovv,
             sin0, sin1, sout0, sout1):
    wid = lax.axis_index("s") * NC + lax.axis_index("c")

    pltpu.sync_copy(sel_hbm, selv)
    pltpu.sync_copy(ov_hbm, ovv)
    sel_vecs = [selv[pl.ds(16 * g, 16)] for g in range(NGRP)]
    ov_vecs = [ovv[pl.ds(16 * g, 16)] for g in range(NGRP)]

    lane = lax.iota(jnp.int32, 16)
    col_vecs = [lane + (16 * g) for g in range(NGRP)]

    in_bufs = [in0, in1]
    out_bufs = [out0, out1]
    sins = [sin0, sin1]
    souts = [sout0, sout1]

    def cid(k):
        return wid + NW * k

    def in_copy(k, b):
        return pltpu.make_async_copy(
            x_hbm.at[pl.ds(cid(k) * CHUNK, CHUNK)], in_bufs[b], sins[b])

    def out_copy(k, b):
        return pltpu.make_async_copy(
            out_bufs[b],
            out_hbm.at[pl.ds(cid(k) * CHUNK, CHUNK)], souts[b])

    def when_valid(k, fn):
        if (k + 1) * NW <= NCHUNK:
            fn()
        else:
            pl.when(cid(k) < NCHUNK)(fn)

    when_valid(0, lambda: in_copy(0, 0).start())
    when_valid(1, lambda: in_copy(1, 1).start())

    for k in range(KMAX):
        b = k & 1

        def step(k=k, b=b):
            in_copy(k, b).wait()
            if k >= 2:
                out_copy(k - 2, b).wait()
            ibuf = in_bufs[b]
            obuf = out_bufs[b]

            @plsc.parallel_loop(0, CHUNK, unroll=4)
            def _rows(r):
                rvec = jnp.full((16,), r, jnp.int32)
                for g in range(NGRP):
                    vals = plsc.load_gather(ibuf, [rvec, sel_vecs[g]])
                    plsc.store_scatter(obuf, [rvec, col_vecs[g]],
                                       vals * ov_vecs[g])

            out_copy(k, b).start()
            if k + 2 < KMAX:
                when_valid(k + 2, lambda: in_copy(k + 2, b).start())

        when_valid(k, step)

    when_valid(KMAX - 2, lambda: out_copy(KMAX - 2, (KMAX - 2) & 1).wait())
    when_valid(KMAX - 1, lambda: out_copy(KMAX - 1, (KMAX - 1) & 1).wait())


_sc_call = pl.kernel(
    _sc_body,
    out_type=jax.ShapeDtypeStruct((NB_ROWS, NZ), jnp.float32),
    mesh=plsc.VectorSubcoreMesh(
        core_axis_name="c", subcore_axis_name="s",
        num_cores=NC, num_subcores=NS),
    scratch_types=[
        pltpu.VMEM((CHUNK, IN_DIM), jnp.float32),
        pltpu.VMEM((CHUNK, IN_DIM), jnp.float32),
        pltpu.VMEM((CHUNK, NZ), jnp.float32),
        pltpu.VMEM((CHUNK, NZ), jnp.float32),
        pltpu.VMEM((NZ,), jnp.int32),
        pltpu.VMEM((NZ,), jnp.float32),
        pltpu.SemaphoreType.DMA,
        pltpu.SemaphoreType.DMA,
        pltpu.SemaphoreType.DMA,
        pltpu.SemaphoreType.DMA,
    ],
    compiler_params=pltpu.CompilerParams(needs_layout_passes=False),
)


def _tca_body(sel_ref, ov_ref, xt_ref, o_ref):
    onehot = (sel_ref[...][:, None]
              == lax.broadcasted_iota(jnp.int32, (NZ, IN_DIM), 1))
    w = onehot.astype(jnp.float32) * ov_ref[...][:, None]
    o_ref[0:NZ, :] = jnp.dot(w, xt_ref[...],
                             preferred_element_type=jnp.float32)
    o_ref[NZ:F_DIM, :] = jnp.zeros((F_DIM - NZ, NBT), jnp.float32)


_tc_a = pl.pallas_call(
    _tca_body,
    grid=(NBLK_A,),
    in_specs=[
        pl.BlockSpec((NZ,), lambda i: (0,)),
        pl.BlockSpec((NZ,), lambda i: (0,)),
        pl.BlockSpec((IN_DIM, NBT), lambda i: (0, i)),
    ],
    out_specs=pl.BlockSpec((F_DIM, NBT), lambda i: (0, i)),
    out_shape=jax.ShapeDtypeStruct((F_DIM, N), jnp.float32),
)


def _tcb_body(buf_ref, sc_ref, o_ref):
    del buf_ref
    o_ref[0:NZ, :] = sc_ref[...].T
    o_ref[NZ:F_DIM, :] = jnp.zeros((F_DIM - NZ, NBT), jnp.float32)


_tc_b = pl.pallas_call(
    _tcb_body,
    grid=(NBLK_B,),
    in_specs=[
        pl.BlockSpec(memory_space=pl.ANY),
        pl.BlockSpec((NBT, NZ), lambda i: (i, 0)),
    ],
    out_specs=pl.BlockSpec((F_DIM, NBT), lambda i: (0, i + NBLK_A)),
    out_shape=jax.ShapeDtypeStruct((F_DIM, N), jnp.float32),
    input_output_aliases={0: 0},
)


@jax.jit
def kernel(charge_density, overlap_constants, select_indices):
    sel = select_indices.astype(jnp.int32)
    ov = overlap_constants.astype(jnp.float32)
    sc_out = _sc_call(charge_density[NA_ROWS:], sel, ov)
    buf = _tc_a(sel, ov, charge_density.T)
    out_t = _tc_b(buf, sc_out)
    return out_t.T


# speedup vs baseline: 7.9645x; 1.0789x over previous
import jax
import jax.numpy as jnp
from jax import lax
from jax.experimental import pallas as pl
from jax.experimental.pallas import tpu as pltpu
from jax.experimental.pallas import tpu_sc as plsc

N = 100000
IN_DIM = 16
F_DIM = 200
NZ = 128
NC = 2
NS = 16
NW = NC * NS

NBT_A = 2560
NBLK_A = 27
NA_ROWS = NBLK_A * NBT_A
NB_ROWS = N - NA_ROWS
NBT = 1280
NBLK_B = -(-NB_ROWS // NBT)

CHUNK = 160
NCHUNK = NB_ROWS // CHUNK
KMAX = -(-NCHUNK // NW)
NGRP = NZ // 16


def _sc_body(x_hbm, sel_hbm, ov_hbm, out_hbm,
             in0, in1, out0, out1, selv, ovv,
             sin0, sin1, sout0, sout1):
    wid = lax.axis_index("s") * NC + lax.axis_index("c")

    pltpu.sync_copy(sel_hbm, selv)
    pltpu.sync_copy(ov_hbm, ovv)
    sel_vecs = [selv[pl.ds(16 * g, 16)] for g in range(NGRP)]
    ov_vecs = [ovv[pl.ds(16 * g, 16)] for g in range(NGRP)]

    lane = lax.iota(jnp.int32, 16)
    col_vecs = [lane + (16 * g) for g in range(NGRP)]

    in_bufs = [in0, in1]
    out_bufs = [out0, out1]
    sins = [sin0, sin1]
    souts = [sout0, sout1]

    def cid(k):
        return wid + NW * k

    def in_copy(k, b):
        return pltpu.make_async_copy(
            x_hbm.at[pl.ds(cid(k) * CHUNK, CHUNK)], in_bufs[b], sins[b])

    def out_copy(k, b):
        return pltpu.make_async_copy(
            out_bufs[b],
            out_hbm.at[pl.ds(cid(k) * CHUNK, CHUNK)], souts[b])

    def when_valid(k, fn):
        if (k + 1) * NW <= NCHUNK:
            fn()
        else:
            pl.when(cid(k) < NCHUNK)(fn)

    when_valid(0, lambda: in_copy(0, 0).start())
    when_valid(1, lambda: in_copy(1, 1).start())

    for k in range(KMAX):
        b = k & 1

        def step(k=k, b=b):
            in_copy(k, b).wait()
            if k >= 2:
                out_copy(k - 2, b).wait()
            ibuf = in_bufs[b]
            obuf = out_bufs[b]

            @plsc.parallel_loop(0, CHUNK, unroll=4)
            def _rows(r):
                rvec = jnp.full((16,), r, jnp.int32)
                for g in range(NGRP):
                    vals = plsc.load_gather(ibuf, [rvec, sel_vecs[g]])
                    plsc.store_scatter(obuf, [rvec, col_vecs[g]],
                                       vals * ov_vecs[g])

            out_copy(k, b).start()
            if k + 2 < KMAX:
                when_valid(k + 2, lambda: in_copy(k + 2, b).start())

        when_valid(k, step)

    when_valid(KMAX - 2, lambda: out_copy(KMAX - 2, (KMAX - 2) & 1).wait())
    when_valid(KMAX - 1, lambda: out_copy(KMAX - 1, (KMAX - 1) & 1).wait())


_sc_call = pl.kernel(
    _sc_body,
    out_type=jax.ShapeDtypeStruct((NB_ROWS, NZ), jnp.float32),
    mesh=plsc.VectorSubcoreMesh(
        core_axis_name="c", subcore_axis_name="s",
        num_cores=NC, num_subcores=NS),
    scratch_types=[
        pltpu.VMEM((CHUNK, IN_DIM), jnp.float32),
        pltpu.VMEM((CHUNK, IN_DIM), jnp.float32),
        pltpu.VMEM((CHUNK, NZ), jnp.float32),
        pltpu.VMEM((CHUNK, NZ), jnp.float32),
        pltpu.VMEM((NZ,), jnp.int32),
        pltpu.VMEM((NZ,), jnp.float32),
        pltpu.SemaphoreType.DMA,
        pltpu.SemaphoreType.DMA,
        pltpu.SemaphoreType.DMA,
        pltpu.SemaphoreType.DMA,
    ],
    compiler_params=pltpu.CompilerParams(needs_layout_passes=False),
)


def _tca_body(sel_ref, ov_ref, xt_ref, o_ref):
    onehot = (sel_ref[...][:, None]
              == lax.broadcasted_iota(jnp.int32, (NZ, IN_DIM), 1))
    w = onehot.astype(jnp.float32) * ov_ref[...][:, None]
    o_ref[0:NZ, :] = jnp.dot(w, xt_ref[...],
                             precision=lax.Precision.HIGHEST,
                             preferred_element_type=jnp.float32)
    o_ref[NZ:F_DIM, :] = jnp.zeros((F_DIM - NZ, NBT_A), jnp.float32)


_tc_a = pl.pallas_call(
    _tca_body,
    grid=(NBLK_A,),
    in_specs=[
        pl.BlockSpec((NZ,), lambda i: (0,)),
        pl.BlockSpec((NZ,), lambda i: (0,)),
        pl.BlockSpec((IN_DIM, NBT_A), lambda i: (0, i)),
    ],
    out_specs=pl.BlockSpec((F_DIM, NBT_A), lambda i: (0, i)),
    out_shape=jax.ShapeDtypeStruct((F_DIM, N), jnp.float32),
)


def _tcb_body(buf_ref, sc_ref, o_ref):
    del buf_ref
    o_ref[0:NZ, :] = sc_ref[...].T
    o_ref[NZ:F_DIM, :] = jnp.zeros((F_DIM - NZ, NBT), jnp.float32)


_tc_b = pl.pallas_call(
    _tcb_body,
    grid=(NBLK_B,),
    in_specs=[
        pl.BlockSpec(memory_space=pl.ANY),
        pl.BlockSpec((NBT, NZ), lambda i: (i, 0)),
    ],
    out_specs=pl.BlockSpec((F_DIM, NBT),
                           lambda i: (0, i + NA_ROWS // NBT)),
    out_shape=jax.ShapeDtypeStruct((F_DIM, N), jnp.float32),
    input_output_aliases={0: 0},
)


@jax.jit
def kernel(charge_density, overlap_constants, select_indices):
    sel = select_indices.astype(jnp.int32)
    ov = overlap_constants.astype(jnp.float32)
    sc_out = _sc_call(charge_density[NA_ROWS:], sel, ov)
    buf = _tc_a(sel, ov, charge_density.T)
    out_t = _tc_b(buf, sc_out)
    return out_t.T


# speedup vs baseline: 8.8705x; 1.1138x over previous
import jax
import jax.numpy as jnp
from jax import lax
from jax.experimental import pallas as pl
from jax.experimental.pallas import tpu as pltpu
from jax.experimental.pallas import tpu_sc as plsc

N = 100000
IN_DIM = 16
F_DIM = 200
NZ = 128
NC = 2
NS = 16
NW = NC * NS

NBT_A = 3840
NBLK_A = 18
NA_ROWS = NBLK_A * NBT_A
NB_ROWS = N - NA_ROWS
NBT = 2560
NBLK_B = -(-NB_ROWS // NBT)

CHUNK = 160
NCHUNK = NB_ROWS // CHUNK
KMAX = -(-NCHUNK // NW)
NGRP = NZ // 16


def _sc_body(x_hbm, sel_hbm, ov_hbm, out_hbm,
             in0, in1, out0, out1, selv, ovv,
             sin0, sin1, sout0, sout1):
    wid = lax.axis_index("s") * NC + lax.axis_index("c")

    pltpu.sync_copy(sel_hbm, selv)
    pltpu.sync_copy(ov_hbm, ovv)
    sel_vecs = [selv[pl.ds(16 * g, 16)] for g in range(NGRP)]
    ov_vecs = [ovv[pl.ds(16 * g, 16)] for g in range(NGRP)]

    lane = lax.iota(jnp.int32, 16)
    col_vecs = [lane + (16 * g) for g in range(NGRP)]

    in_bufs = [in0, in1]
    out_bufs = [out0, out1]
    sins = [sin0, sin1]
    souts = [sout0, sout1]

    def cid(k):
        return wid + NW * k

    def in_copy(k, b):
        return pltpu.make_async_copy(
            x_hbm.at[pl.ds(cid(k) * CHUNK, CHUNK)], in_bufs[b], sins[b])

    def out_copy(k, b):
        return pltpu.make_async_copy(
            out_bufs[b],
            out_hbm.at[pl.ds(cid(k) * CHUNK, CHUNK)], souts[b])

    def when_valid(k, fn):
        if (k + 1) * NW <= NCHUNK:
            fn()
        else:
            pl.when(cid(k) < NCHUNK)(fn)

    when_valid(0, lambda: in_copy(0, 0).start())
    when_valid(1, lambda: in_copy(1, 1).start())

    for k in range(KMAX):
        b = k & 1

        def step(k=k, b=b):
            in_copy(k, b).wait()
            if k >= 2:
                out_copy(k - 2, b).wait()
            ibuf = in_bufs[b]
            obuf = out_bufs[b]

            @plsc.parallel_loop(0, CHUNK, unroll=4)
            def _rows(r):
                rvec = jnp.full((16,), r, jnp.int32)
                for g in range(NGRP):
                    vals = plsc.load_gather(ibuf, [rvec, sel_vecs[g]])
                    plsc.store_scatter(obuf, [rvec, col_vecs[g]],
                                       vals * ov_vecs[g])

            out_copy(k, b).start()
            if k + 2 < KMAX:
                when_valid(k + 2, lambda: in_copy(k + 2, b).start())

        when_valid(k, step)

    when_valid(KMAX - 2, lambda: out_copy(KMAX - 2, (KMAX - 2) & 1).wait())
    when_valid(KMAX - 1, lambda: out_copy(KMAX - 1, (KMAX - 1) & 1).wait())


_sc_call = pl.kernel(
    _sc_body,
    out_type=jax.ShapeDtypeStruct((NB_ROWS, NZ), jnp.float32),
    mesh=plsc.VectorSubcoreMesh(
        core_axis_name="c", subcore_axis_name="s",
        num_cores=NC, num_subcores=NS),
    scratch_types=[
        pltpu.VMEM((CHUNK, IN_DIM), jnp.float32),
        pltpu.VMEM((CHUNK, IN_DIM), jnp.float32),
        pltpu.VMEM((CHUNK, NZ), jnp.float32),
        pltpu.VMEM((CHUNK, NZ), jnp.float32),
        pltpu.VMEM((NZ,), jnp.int32),
        pltpu.VMEM((NZ,), jnp.float32),
        pltpu.SemaphoreType.DMA,
        pltpu.SemaphoreType.DMA,
        pltpu.SemaphoreType.DMA,
        pltpu.SemaphoreType.DMA,
    ],
    compiler_params=pltpu.CompilerParams(needs_layout_passes=False),
)


def _tca_body(sel_ref, ov_ref, xt_ref, o_ref):
    onehot = (sel_ref[...][:, None]
              == lax.broadcasted_iota(jnp.int32, (NZ, IN_DIM), 1))
    w = onehot.astype(jnp.float32) * ov_ref[...][:, None]
    o_ref[0:NZ, :] = jnp.dot(w, xt_ref[...],
                             precision=lax.Precision.HIGHEST,
                             preferred_element_type=jnp.float32)
    o_ref[NZ:F_DIM, :] = jnp.zeros((F_DIM - NZ, NBT_A), jnp.float32)


_tc_a = pl.pallas_call(
    _tca_body,
    grid=(NBLK_A,),
    in_specs=[
        pl.BlockSpec((NZ,), lambda i: (0,)),
        pl.BlockSpec((NZ,), lambda i: (0,)),
        pl.BlockSpec((IN_DIM, NBT_A), lambda i: (0, i)),
    ],
    out_specs=pl.BlockSpec((F_DIM, NBT_A), lambda i: (0, i)),
    out_shape=jax.ShapeDtypeStruct((F_DIM, N), jnp.float32),
)


def _tcb_body(buf_ref, sc_ref, o_ref):
    del buf_ref
    o_ref[0:NZ, :] = sc_ref[...].T
    o_ref[NZ:F_DIM, :] = jnp.zeros((F_DIM - NZ, NBT), jnp.float32)


_tc_b = pl.pallas_call(
    _tcb_body,
    grid=(NBLK_B,),
    in_specs=[
        pl.BlockSpec(memory_space=pl.ANY),
        pl.BlockSpec((NBT, NZ), lambda i: (i, 0)),
    ],
    out_specs=pl.BlockSpec((F_DIM, NBT),
                           lambda i: (0, i + NA_ROWS // NBT)),
    out_shape=jax.ShapeDtypeStruct((F_DIM, N), jnp.float32),
    input_output_aliases={0: 0},
)


@jax.jit
def kernel(charge_density, overlap_constants, select_indices):
    sel = select_indices.astype(jnp.int32)
    ov = overlap_constants.astype(jnp.float32)
    sc_out = _sc_call(charge_density[NA_ROWS:], sel, ov)
    buf = _tc_a(sel, ov, charge_density.T)
    out_t = _tc_b(buf, sc_out)
    return out_t.T


# speedup vs baseline: 9.3638x; 1.0556x over previous
import jax
import jax.numpy as jnp
from jax import lax
from jax.experimental import pallas as pl
from jax.experimental.pallas import tpu as pltpu
from jax.experimental.pallas import tpu_sc as plsc

N = 100000
IN_DIM = 16
F_DIM = 200
NZ = 128
NC = 2
NS = 16
NW = NC * NS

NBT_A = 7680
NBLK_A = 9
NA_ROWS = NBLK_A * NBT_A
NB_ROWS = N - NA_ROWS
NBT = 4608
NBLK_B = -(-NB_ROWS // NBT)

CHUNK = 160
NCHUNK = NB_ROWS // CHUNK
KMAX = -(-NCHUNK // NW)
NGRP = NZ // 16


def _sc_body(x_hbm, sel_hbm, ov_hbm, out_hbm,
             in0, in1, out0, out1, selv, ovv,
             sin0, sin1, sout0, sout1):
    wid = lax.axis_index("s") * NC + lax.axis_index("c")

    pltpu.sync_copy(sel_hbm, selv)
    pltpu.sync_copy(ov_hbm, ovv)
    sel_vecs = [selv[pl.ds(16 * g, 16)] for g in range(NGRP)]
    ov_vecs = [ovv[pl.ds(16 * g, 16)] for g in range(NGRP)]

    lane = lax.iota(jnp.int32, 16)
    col_vecs = [lane + (16 * g) for g in range(NGRP)]

    in_bufs = [in0, in1]
    out_bufs = [out0, out1]
    sins = [sin0, sin1]
    souts = [sout0, sout1]

    def cid(k):
        return wid + NW * k

    def in_copy(k, b):
        return pltpu.make_async_copy(
            x_hbm.at[pl.ds(cid(k) * CHUNK, CHUNK)], in_bufs[b], sins[b])

    def out_copy(k, b):
        return pltpu.make_async_copy(
            out_bufs[b],
            out_hbm.at[pl.ds(cid(k) * CHUNK, CHUNK)], souts[b])

    def when_valid(k, fn):
        if (k + 1) * NW <= NCHUNK:
            fn()
        else:
            pl.when(cid(k) < NCHUNK)(fn)

    when_valid(0, lambda: in_copy(0, 0).start())
    when_valid(1, lambda: in_copy(1, 1).start())

    for k in range(KMAX):
        b = k & 1

        def step(k=k, b=b):
            in_copy(k, b).wait()
            if k >= 2:
                out_copy(k - 2, b).wait()
            ibuf = in_bufs[b]
            obuf = out_bufs[b]

            @plsc.parallel_loop(0, CHUNK, unroll=4)
            def _rows(r):
                rvec = jnp.full((16,), r, jnp.int32)
                for g in range(NGRP):
                    vals = plsc.load_gather(ibuf, [rvec, sel_vecs[g]])
                    plsc.store_scatter(obuf, [rvec, col_vecs[g]],
                                       vals * ov_vecs[g])

            out_copy(k, b).start()
            if k + 2 < KMAX:
                when_valid(k + 2, lambda: in_copy(k + 2, b).start())

        when_valid(k, step)

    when_valid(KMAX - 2, lambda: out_copy(KMAX - 2, (KMAX - 2) & 1).wait())
    when_valid(KMAX - 1, lambda: out_copy(KMAX - 1, (KMAX - 1) & 1).wait())


_sc_call = pl.kernel(
    _sc_body,
    out_type=jax.ShapeDtypeStruct((NB_ROWS, NZ), jnp.float32),
    mesh=plsc.VectorSubcoreMesh(
        core_axis_name="c", subcore_axis_name="s",
        num_cores=NC, num_subcores=NS),
    scratch_types=[
        pltpu.VMEM((CHUNK, IN_DIM), jnp.float32),
        pltpu.VMEM((CHUNK, IN_DIM), jnp.float32),
        pltpu.VMEM((CHUNK, NZ), jnp.float32),
        pltpu.VMEM((CHUNK, NZ), jnp.float32),
        pltpu.VMEM((NZ,), jnp.int32),
        pltpu.VMEM((NZ,), jnp.float32),
        pltpu.SemaphoreType.DMA,
        pltpu.SemaphoreType.DMA,
        pltpu.SemaphoreType.DMA,
        pltpu.SemaphoreType.DMA,
    ],
    compiler_params=pltpu.CompilerParams(needs_layout_passes=False),
)


def _tca_body(sel_ref, ov_ref, xt_ref, o_ref):
    onehot = (sel_ref[...][:, None]
              == lax.broadcasted_iota(jnp.int32, (NZ, IN_DIM), 1))
    w = onehot.astype(jnp.float32) * ov_ref[...][:, None]
    o_ref[0:NZ, :] = jnp.dot(w, xt_ref[...],
                             precision=lax.Precision.HIGHEST,
                             preferred_element_type=jnp.float32)
    o_ref[NZ:F_DIM, :] = jnp.zeros((F_DIM - NZ, NBT_A), jnp.float32)


_tc_a = pl.pallas_call(
    _tca_body,
    grid=(NBLK_A,),
    in_specs=[
        pl.BlockSpec((NZ,), lambda i: (0,)),
        pl.BlockSpec((NZ,), lambda i: (0,)),
        pl.BlockSpec((IN_DIM, NBT_A), lambda i: (0, i)),
    ],
    out_specs=pl.BlockSpec((F_DIM, NBT_A), lambda i: (0, i)),
    out_shape=jax.ShapeDtypeStruct((F_DIM, N), jnp.float32),
)


def _tcb_body(buf_ref, sc_ref, o_ref):
    del buf_ref
    o_ref[0:NZ, :] = sc_ref[...].T
    o_ref[NZ:F_DIM, :] = jnp.zeros((F_DIM - NZ, NBT), jnp.float32)


_tc_b = pl.pallas_call(
    _tcb_body,
    grid=(NBLK_B,),
    in_specs=[
        pl.BlockSpec(memory_space=pl.ANY),
        pl.BlockSpec((NBT, NZ), lambda i: (i, 0)),
    ],
    out_specs=pl.BlockSpec((F_DIM, NBT),
                           lambda i: (0, i + NA_ROWS // NBT)),
    out_shape=jax.ShapeDtypeStruct((F_DIM, N), jnp.float32),
    input_output_aliases={0: 0},
)


@jax.jit
def kernel(charge_density, overlap_constants, select_indices):
    sel = select_indices.astype(jnp.int32)
    ov = overlap_constants.astype(jnp.float32)
    sc_out = _sc_call(charge_density[NA_ROWS:], sel, ov)
    buf = _tc_a(sel, ov, charge_density.T)
    out_t = _tc_b(buf, sc_out)
    return out_t.T


# speedup vs baseline: 9.3924x; 1.0031x over previous
import jax
import jax.numpy as jnp
from jax import lax
from jax.experimental import pallas as pl
from jax.experimental.pallas import tpu as pltpu
from jax.experimental.pallas import tpu_sc as plsc

N = 100000
IN_DIM = 16
F_DIM = 200
NZ = 128
NC = 2
NS = 16
NW = NC * NS

NBT_A = 11520
NBLK_A = 6
NA_ROWS = NBLK_A * NBT_A
NB_ROWS = N - NA_ROWS
NBT = 4608
NBLK_B = -(-NB_ROWS // NBT)

CHUNK = 160
NCHUNK = NB_ROWS // CHUNK
KMAX = -(-NCHUNK // NW)
NGRP = NZ // 16


def _sc_body(x_hbm, sel_hbm, ov_hbm, out_hbm,
             in0, in1, out0, out1, selv, ovv,
             sin0, sin1, sout0, sout1):
    wid = lax.axis_index("s") * NC + lax.axis_index("c")

    pltpu.sync_copy(sel_hbm, selv)
    pltpu.sync_copy(ov_hbm, ovv)
    sel_vecs = [selv[pl.ds(16 * g, 16)] for g in range(NGRP)]
    ov_vecs = [ovv[pl.ds(16 * g, 16)] for g in range(NGRP)]

    lane = lax.iota(jnp.int32, 16)
    col_vecs = [lane + (16 * g) for g in range(NGRP)]

    in_bufs = [in0, in1]
    out_bufs = [out0, out1]
    sins = [sin0, sin1]
    souts = [sout0, sout1]

    def cid(k):
        return wid + NW * k

    def in_copy(k, b):
        return pltpu.make_async_copy(
            x_hbm.at[pl.ds(cid(k) * CHUNK, CHUNK)], in_bufs[b], sins[b])

    def out_copy(k, b):
        return pltpu.make_async_copy(
            out_bufs[b],
            out_hbm.at[pl.ds(cid(k) * CHUNK, CHUNK)], souts[b])

    def when_valid(k, fn):
        if (k + 1) * NW <= NCHUNK:
            fn()
        else:
            pl.when(cid(k) < NCHUNK)(fn)

    when_valid(0, lambda: in_copy(0, 0).start())
    when_valid(1, lambda: in_copy(1, 1).start())

    for k in range(KMAX):
        b = k & 1

        def step(k=k, b=b):
            in_copy(k, b).wait()
            if k >= 2:
                out_copy(k - 2, b).wait()
            ibuf = in_bufs[b]
            obuf = out_bufs[b]

            @plsc.parallel_loop(0, CHUNK, unroll=4)
            def _rows(r):
                rvec = jnp.full((16,), r, jnp.int32)
                for g in range(NGRP):
                    vals = plsc.load_gather(ibuf, [rvec, sel_vecs[g]])
                    plsc.store_scatter(obuf, [rvec, col_vecs[g]],
                                       vals * ov_vecs[g])

            out_copy(k, b).start()
            if k + 2 < KMAX:
                when_valid(k + 2, lambda: in_copy(k + 2, b).start())

        when_valid(k, step)

    when_valid(KMAX - 2, lambda: out_copy(KMAX - 2, (KMAX - 2) & 1).wait())
    when_valid(KMAX - 1, lambda: out_copy(KMAX - 1, (KMAX - 1) & 1).wait())


_sc_call = pl.kernel(
    _sc_body,
    out_type=jax.ShapeDtypeStruct((NB_ROWS, NZ), jnp.float32),
    mesh=plsc.VectorSubcoreMesh(
        core_axis_name="c", subcore_axis_name="s",
        num_cores=NC, num_subcores=NS),
    scratch_types=[
        pltpu.VMEM((CHUNK, IN_DIM), jnp.float32),
        pltpu.VMEM((CHUNK, IN_DIM), jnp.float32),
        pltpu.VMEM((CHUNK, NZ), jnp.float32),
        pltpu.VMEM((CHUNK, NZ), jnp.float32),
        pltpu.VMEM((NZ,), jnp.int32),
        pltpu.VMEM((NZ,), jnp.float32),
        pltpu.SemaphoreType.DMA,
        pltpu.SemaphoreType.DMA,
        pltpu.SemaphoreType.DMA,
        pltpu.SemaphoreType.DMA,
    ],
    compiler_params=pltpu.CompilerParams(needs_layout_passes=False),
)


def _tca_body(sel_ref, ov_ref, xt_ref, o_ref):
    onehot = (sel_ref[...][:, None]
              == lax.broadcasted_iota(jnp.int32, (NZ, IN_DIM), 1))
    w = onehot.astype(jnp.float32) * ov_ref[...][:, None]
    o_ref[0:NZ, :] = jnp.dot(w, xt_ref[...],
                             precision=lax.Precision.HIGHEST,
                             preferred_element_type=jnp.float32)
    o_ref[NZ:F_DIM, :] = jnp.zeros((F_DIM - NZ, NBT_A), jnp.float32)


_tc_a = pl.pallas_call(
    _tca_body,
    grid=(NBLK_A,),
    in_specs=[
        pl.BlockSpec((NZ,), lambda i: (0,)),
        pl.BlockSpec((NZ,), lambda i: (0,)),
        pl.BlockSpec((IN_DIM, NBT_A), lambda i: (0, i)),
    ],
    out_specs=pl.BlockSpec((F_DIM, NBT_A), lambda i: (0, i)),
    out_shape=jax.ShapeDtypeStruct((F_DIM, N), jnp.float32),
)


def _tcb_body(buf_ref, sc_ref, o_ref):
    del buf_ref
    o_ref[0:NZ, :] = sc_ref[...].T
    o_ref[NZ:F_DIM, :] = jnp.zeros((F_DIM - NZ, NBT), jnp.float32)


_tc_b = pl.pallas_call(
    _tcb_body,
    grid=(NBLK_B,),
    in_specs=[
        pl.BlockSpec(memory_space=pl.ANY),
        pl.BlockSpec((NBT, NZ), lambda i: (i, 0)),
    ],
    out_specs=pl.BlockSpec((F_DIM, NBT),
                           lambda i: (0, i + NA_ROWS // NBT)),
    out_shape=jax.ShapeDtypeStruct((F_DIM, N), jnp.float32),
    input_output_aliases={0: 0},
)


@jax.jit
def kernel(charge_density, overlap_constants, select_indices):
    sel = select_indices.astype(jnp.int32)
    ov = overlap_constants.astype(jnp.float32)
    sc_out = _sc_call(charge_density[NA_ROWS:], sel, ov)
    buf = _tc_a(sel, ov, charge_density.T)
    out_t = _tc_b(buf, sc_out)
    return out_t.T
